# baseline (speedup 1.0000x reference)
import functools

import jax
import jax.numpy as jnp
from jax import lax
from jax.experimental import pallas as pl
from jax.experimental.pallas import tpu as pltpu
from jax.experimental.pallas import tpu_sc as plsc

_NC = 2
_NS = 16
_NW = _NC * _NS
_R = 32


def _sc_body(x_ref, cls_ref, pos_ref, out_ref,
             xbufs, posbufs, pidx, oidx, cbuf, pcls, cidx,
             in_sems, out_sems, pos_sems, csem,
             *, batch, seq_len, d, r):
    c = lax.axis_index("c")
    s = lax.axis_index("s")
    wid = s * _NC + c
    rows_pw = seq_len // _NW
    chunks_pb = rows_pw // r
    total = chunks_pb * batch

    iota16 = lax.iota(jnp.int32, 16)

    def fill_idx(idx, slot, base):
        for q in range(r // 16):
            idx[slot, pl.ds(q * 16, 16)] = iota16 + (base + q * 16)

    def start_pos(j):
        pslot = lax.rem(j, 2)
        base = 1 + wid * rows_pw + j * r
        fill_idx(pidx, pslot, base)
        pltpu.async_copy(
            pos_ref.at[pidx.at[pslot]], posbufs.at[pslot], pos_sems.at[pslot]
        )

    def wait_pos(j):
        pslot = lax.rem(j, 2)
        pltpu.make_async_copy(
            pos_ref.at[pl.ds(0, r), :], posbufs.at[pslot], pos_sems.at[pslot]
        ).wait()

    def start_in(i):
        j, b = i // batch, i % batch
        slot = lax.rem(i, 2)
        x_row0 = b * seq_len + wid * rows_pw + j * r
        pltpu.async_copy(
            x_ref.at[pl.ds(x_row0, r), :], xbufs.at[slot], in_sems.at[slot]
        )

    def wait_in(i):
        slot = lax.rem(i, 2)
        pltpu.make_async_copy(
            x_ref.at[pl.ds(0, r), :], xbufs.at[slot], in_sems.at[slot]
        ).wait()

    def start_out(i):
        j, b = i // batch, i % batch
        slot = lax.rem(i, 2)
        base = b * (seq_len + 1) + 1 + wid * rows_pw + j * r
        fill_idx(oidx, slot, base)
        pltpu.async_copy(
            xbufs.at[slot], out_ref.at[oidx.at[slot]], out_sems.at[slot]
        )

    def wait_out(i):
        slot = lax.rem(i, 2)
        pltpu.make_async_copy(
            x_ref.at[pl.ds(0, r), :], xbufs.at[slot], out_sems.at[slot]
        ).wait()

    def tec_add(i):
        slot = lax.rem(i, 2)
        pslot = lax.rem(i // batch, 2)

        def row_body(row, _):
            for q in range(d // 16):
                sl = pl.ds(q * 16, 16)
                xbufs[slot, row, sl] = xbufs[slot, row, sl] + posbufs[pslot, row, sl]
            return ()

        lax.fori_loop(0, r, row_body, ())

    @pl.when(wid == 0)
    def _():
        cidx[...] = iota16 * 0
        pltpu.async_copy(cls_ref.at[cidx], cbuf, csem).wait()
        pltpu.async_copy(pos_ref.at[cidx], pcls, csem).wait()

        def crow_body(row, _):
            for q in range(d // 16):
                sl = pl.ds(q * 16, 16)
                cbuf[row, sl] = cbuf[row, sl] + pcls[row, sl]
            return ()

        lax.fori_loop(0, 16, crow_body, ())
        cidx[...] = (iota16 & 3) * (seq_len + 1)
        pltpu.async_copy(cbuf, out_ref.at[cidx], csem)

    start_pos(0)
    start_in(0)

    def step(i, _):
        j, b = i // batch, i % batch

        @pl.when(b == 0)
        def _():
            wait_pos(j)

            @pl.when(j + 1 < chunks_pb)
            def _():
                start_pos(j + 1)

        @pl.when(i + 1 < total)
        def _():
            @pl.when(i >= 1)
            def _():
                wait_out(i - 1)

            start_in(i + 1)

        wait_in(i)
        tec_add(i)
        start_out(i)
        return ()

    lax.fori_loop(0, total, step, ())
    wait_out(total - 1)
    wait_out(total - 2)

    @pl.when(wid == 0)
    def _():
        pltpu.make_async_copy(cbuf, out_ref.at[cidx], csem).wait()


def kernel(x, cls_token, pos_table):
    batch, seq_len, d = x.shape
    r = _R

    mesh = plsc.VectorSubcoreMesh(
        core_axis_name="c", subcore_axis_name="s",
        num_cores=_NC, num_subcores=_NS,
    )
    sc_fn = pl.kernel(
        functools.partial(_sc_body, batch=batch, seq_len=seq_len, d=d, r=r),
        out_type=jax.ShapeDtypeStruct((batch * (seq_len + 1), d), x.dtype),
        mesh=mesh,
        scratch_types=[
            pltpu.VMEM((2, r, d), x.dtype),
            pltpu.VMEM((2, r, d), x.dtype),
            pltpu.VMEM((2, r), jnp.int32),
            pltpu.VMEM((2, r), jnp.int32),
            pltpu.VMEM((16, d), x.dtype),
            pltpu.VMEM((16, d), x.dtype),
            pltpu.VMEM((16,), jnp.int32),
            pltpu.SemaphoreType.DMA((2,)),
            pltpu.SemaphoreType.DMA((2,)),
            pltpu.SemaphoreType.DMA((2,)),
            pltpu.SemaphoreType.DMA,
        ],
    )
    out = sc_fn(
        x.reshape(batch * seq_len, d),
        cls_token.reshape(1, d),
        pos_table,
    )
    return out.reshape(batch, seq_len + 1, d)


# speedup vs baseline: 4.6604x; 4.6604x over previous
import functools

import jax
import jax.numpy as jnp
from jax.experimental import pallas as pl
from jax.experimental.pallas import tpu as pltpu

_S = 1024
_NBUF = 4


def _body(x_ref, cls_ref, pos_ref, out_ref,
          in_bufs, out_bufs, pos_vmem, halo, tail_buf,
          in_sems, out_sems, pos_sems, tail_sem,
          *, batch, seq_len, d, s, nbuf):
    kx = seq_len // s
    steps = kx * batch

    def in_dma(step):
        k = step // batch
        b = step % batch
        slot = jax.lax.rem(step, nbuf)
        return pltpu.make_async_copy(
            x_ref.at[b, pl.ds(k * s, s), :],
            in_bufs.at[slot],
            in_sems.at[slot],
        )

    def out_dma(step):
        k = step // batch
        b = step % batch
        slot = jax.lax.rem(step, nbuf)
        return pltpu.make_async_copy(
            out_bufs.at[slot],
            out_ref.at[b, pl.ds(k * s, s), :],
            out_sems.at[slot],
        )

    def pos_dma(k):
        return pltpu.make_async_copy(
            pos_ref.at[pl.ds(k * s, s), :],
            pos_vmem.at[pl.ds(k * s, s), :],
            pos_sems.at[k],
        )

    for k in range(kx):
        pos_dma(k).start()
    pltpu.make_async_copy(
        pos_ref.at[pl.ds(kx * s, 1), :], tail_buf, tail_sem
    ).start()
    for i in range(nbuf - 1):
        in_dma(i).start()

    def step_fn(step, _):
        k = step // batch
        b = step % batch
        slot = jax.lax.rem(step, nbuf)

        @pl.when(step + nbuf - 1 < steps)
        def _():
            in_dma(step + nbuf - 1).start()

        @pl.when(b == 0)
        def _():
            pos_dma(k).wait()

        @pl.when(step >= nbuf)
        def _():
            out_dma(step - nbuf).wait()

        in_dma(step).wait()
        xblk = in_bufs[slot]
        first = jnp.where(k == 0, cls_ref[0], halo[b, 0:1, :])
        rolled = pltpu.roll(xblk, shift=1, axis=0)
        out_bufs[slot] = rolled + pos_vmem[pl.ds(k * s, s), :]
        out_bufs[slot, 0:1, :] = first + pos_vmem[pl.ds(k * s, 1), :]
        halo[b, 0:1, :] = xblk[s - 1 : s, :]
        out_dma(step).start(1)
        return ()

    jax.lax.fori_loop(0, steps, step_fn, (), unroll=False)

    pltpu.make_async_copy(
        pos_ref.at[pl.ds(kx * s, 1), :], tail_buf, tail_sem
    ).wait()
    for b in range(batch):
        halo[b, 0:1, :] = halo[b, 0:1, :] + tail_buf[...]
    for b in range(batch):
        pltpu.make_async_copy(
            halo.at[b], out_ref.at[b, pl.ds(seq_len, 1), :], tail_sem
        ).start()
    for b in range(batch):
        pltpu.make_async_copy(
            halo.at[b], out_ref.at[b, pl.ds(seq_len, 1), :], tail_sem
        ).wait()

    def drain(i, _):
        out_dma(i).wait()
        return ()
    jax.lax.fori_loop(steps - nbuf, steps, drain, (), unroll=False)


def kernel(x, cls_token, pos_table):
    batch, seq_len, d = x.shape
    s = _S
    nbuf = _NBUF
    kx = seq_len // s

    out = pl.pallas_call(
        functools.partial(_body, batch=batch, seq_len=seq_len, d=d, s=s,
                          nbuf=nbuf),
        in_specs=[
            pl.BlockSpec(memory_space=pltpu.MemorySpace.HBM),
            pl.BlockSpec((1, 1, d), lambda: (0, 0, 0)),
            pl.BlockSpec(memory_space=pltpu.MemorySpace.HBM),
        ],
        out_specs=pl.BlockSpec(memory_space=pltpu.MemorySpace.HBM),
        out_shape=jax.ShapeDtypeStruct((batch, seq_len + 1, d), x.dtype),
        scratch_shapes=[
            pltpu.VMEM((nbuf, s, d), x.dtype),
            pltpu.VMEM((nbuf, s, d), x.dtype),
            pltpu.VMEM((seq_len, d), x.dtype),
            pltpu.VMEM((batch, 1, d), x.dtype),
            pltpu.VMEM((1, d), x.dtype),
            pltpu.SemaphoreType.DMA((nbuf,)),
            pltpu.SemaphoreType.DMA((nbuf,)),
            pltpu.SemaphoreType.DMA((kx,)),
            pltpu.SemaphoreType.DMA,
        ],
    )(x, cls_token, pos_table)
    return out
